# baseline (speedup 1.0000x reference)
import functools
import jax
import jax.numpy as jnp
from jax import lax
from jax.experimental import pallas as pl
from jax.experimental.pallas import tpu as pltpu
from jax.experimental.pallas import tpu_sc as plsc

_N = 10000
_E = 320000
_DIN = 128
_H = 64
_NC = 2
_NS = 16
_NW = _NC * _NS
_EPW = _E // _NW
_K = 128
_NCHUNK = 79
_EPWP = _NCHUNK * _K
_NP = 10240
_RPT = _NP // _NS

_mesh = plsc.VectorSubcoreMesh(core_axis_name="c", subcore_axis_name="s")
_sc_params = pltpu.CompilerParams(needs_layout_passes=False,
                                  use_tc_tiling_on_sc=False)


@functools.partial(
    pl.kernel,
    mesh=_mesh,
    compiler_params=_sc_params,
    out_type=(
        jax.ShapeDtypeStruct((_NW, _NCHUNK, _K), jnp.float32),
        jax.ShapeDtypeStruct((_NC, _NP, 16), jnp.float32),
    ),
    scratch_types=[
        pltpu.VMEM((_NP,), jnp.float32),
        pltpu.VMEM((_NP,), jnp.float32),
        pltpu.VMEM((_NP,), jnp.float32),
        pltpu.VMEM((_NCHUNK, _K), jnp.int32),
        pltpu.VMEM((_NCHUNK, _K), jnp.int32),
        pltpu.VMEM((_NCHUNK, _K), jnp.float32),
        pltpu.VMEM((_K, 16), jnp.float32),
        pltpu.VMEM_SHARED((_NP, 16), jnp.float32),
    ],
)
def _sc_prep(posx, posy, posz, dste, srce, zeros16, ones16,
             sq_out, cnt_out, px, py, pz, idx_d, idx_s, sqv, onesv, cnt_sh):
    cid = lax.axis_index("c")
    sid = lax.axis_index("s")
    wid = sid * _NC + cid
    pltpu.sync_copy(posx, px)
    pltpu.sync_copy(posy, py)
    pltpu.sync_copy(posz, pz)
    pltpu.sync_copy(dste.at[wid], idx_d)
    pltpu.sync_copy(srce.at[wid], idx_s)
    pltpu.sync_copy(ones16, onesv)
    pltpu.sync_copy(zeros16.at[pl.ds(sid * _RPT, _RPT)],
                    cnt_sh.at[pl.ds(sid * _RPT, _RPT)])
    plsc.subcore_barrier()

    def chunk(t, carry):
        for g in range(_K // 16):
            i_dst = idx_d[t, pl.ds(g * 16, 16)]
            i_src = idx_s[t, pl.ds(g * 16, 16)]
            dx = plsc.load_gather(px, [i_src]) - plsc.load_gather(px, [i_dst])
            dy = plsc.load_gather(py, [i_src]) - plsc.load_gather(py, [i_dst])
            dz = plsc.load_gather(pz, [i_src]) - plsc.load_gather(pz, [i_dst])
            sqv[t, pl.ds(g * 16, 16)] = dx * dx + dy * dy + dz * dz
        pltpu.sync_copy(onesv, cnt_sh.at[idx_d.at[t]], add=True)
        return carry

    lax.fori_loop(0, _NCHUNK, chunk, 0)
    pltpu.sync_copy(sqv, sq_out.at[wid])
    plsc.subcore_barrier()
    pltpu.sync_copy(cnt_sh.at[pl.ds(sid * _RPT, _RPT)],
                    cnt_out.at[cid, pl.ds(sid * _RPT, _RPT)])


@functools.partial(
    pl.kernel,
    mesh=_mesh,
    compiler_params=_sc_params,
    out_type=jax.ShapeDtypeStruct((_NC, _NP, _H), jnp.float32),
    scratch_types=[
        pltpu.VMEM((_NCHUNK, _K), jnp.int32),
        pltpu.VMEM((2, _K), jnp.int32),
        pltpu.VMEM((2, _K), jnp.float32),
        pltpu.VMEM((2, _K, _H), jnp.float32),
        pltpu.VMEM((2, _K, _H), jnp.float32),
        pltpu.VMEM((2, _K, _H), jnp.float32),
        pltpu.VMEM((_H,), jnp.float32),
        pltpu.VMEM_SHARED((_NP, _H), jnp.float32),
    ] + [pltpu.SemaphoreType.DMA] * 10,
)
def _sc_layer(a_hbm, b_hbm, dste, srce, distr, wc_hbm, zeros64,
              p_out, idx_d, idx_s2, dl2, ai2, bj2, vv2, wcv, u_sh,
              sem_a0, sem_a1,
              sem_b0, sem_b1,
              sem_s0, sem_s1,
              sem_d0, sem_d1,
              sem_i0, sem_i1):
    cid = lax.axis_index("c")
    sid = lax.axis_index("s")
    wid = sid * _NC + cid
    pltpu.sync_copy(dste.at[wid], idx_d)
    pltpu.sync_copy(wc_hbm, wcv)
    pltpu.sync_copy(zeros64.at[pl.ds(sid * _RPT, _RPT)],
                    u_sh.at[pl.ds(sid * _RPT, _RPT)])
    plsc.subcore_barrier()
    wc = [wcv[pl.ds(c * 16, 16)] for c in range(_H // 16)]
    sem_a = (sem_a0, sem_a1)
    sem_b = (sem_b0, sem_b1)
    sem_s = (sem_s0, sem_s1)
    sem_d = (sem_d0, sem_d1)
    sem_i = (sem_i0, sem_i1)

    def issue_dl(t, h):
        pltpu.async_copy(distr.at[wid, t], dl2.at[h], sem_d[h])

    def issue_idx(t, h):
        pltpu.async_copy(srce.at[wid, t], idx_s2.at[h], sem_i[h])

    def wait_idx(t, h):
        pltpu.make_async_copy(srce.at[wid, t], idx_s2.at[h], sem_i[h]).wait()

    def issue_gathers(t, h):
        pltpu.async_copy(a_hbm.at[idx_d.at[t]], ai2.at[h], sem_a[h])
        pltpu.async_copy(b_hbm.at[idx_s2.at[h]], bj2.at[h], sem_b[h])

    def wait_gathers(t, h):
        pltpu.make_async_copy(a_hbm.at[idx_d.at[t]], ai2.at[h], sem_a[h]).wait()
        pltpu.make_async_copy(b_hbm.at[idx_s2.at[h]], bj2.at[h], sem_b[h]).wait()

    def compute(t, h):
        ai = ai2.at[h]
        bj = bj2.at[h]
        vv = vv2.at[h]

        def edge16(g, c2):
            dvec = dl2[h, pl.ds(g * 16, 16)]
            for el in range(16):
                e = g * 16 + el
                for c in range(_H // 16):
                    s = pl.ds(c * 16, 16)
                    vv[e, s] = jnp.maximum(
                        ai[e, s] + bj[e, s] + dvec[el] * wc[c], 0.0)
            return c2

        lax.fori_loop(0, _K // 16, edge16, 0, unroll=True)

    def step(t, h):
        @pl.when(t + 1 < _NCHUNK)
        def _():
            wait_idx(t + 1, 1 - h)
            issue_gathers(t + 1, 1 - h)
        wait_gathers(t, h)
        @pl.when(t + 2 < _NCHUNK)
        def _():
            issue_idx(t + 2, h)
        @pl.when(t >= 2)
        def _():
            pltpu.make_async_copy(vv2.at[h], u_sh.at[idx_d.at[t]],
                                  sem_s[h]).wait()
        pltpu.make_async_copy(distr.at[wid, t], dl2.at[h], sem_d[h]).wait()
        compute(t, h)
        pltpu.async_copy(vv2.at[h], u_sh.at[idx_d.at[t]], sem_s[h], add=True)
        @pl.when(t + 2 < _NCHUNK)
        def _():
            issue_dl(t + 2, h)

    issue_dl(0, 0)
    issue_dl(1, 1)
    issue_idx(0, 0)
    issue_idx(1, 1)
    wait_idx(0, 0)
    issue_gathers(0, 0)
    step(0, 0)

    def pair(tt, carry):
        step(2 * tt + 1, 1)
        step(2 * tt + 2, 0)
        return carry

    lax.fori_loop(0, (_NCHUNK - 1) // 2, pair, 0)
    pltpu.make_async_copy(vv2.at[0], u_sh.at[idx_d.at[0]], sem_s0).wait()
    pltpu.make_async_copy(vv2.at[1], u_sh.at[idx_d.at[0]], sem_s1).wait()
    plsc.subcore_barrier()
    pltpu.sync_copy(u_sh.at[pl.ds(sid * _RPT, _RPT)],
                    p_out.at[cid, pl.ds(sid * _RPT, _RPT)])


def _tc_embed_body(x_ref, wemb_ref, bemb_ref, w1ab_ref, b1_ref, sq_ref,
                   cntp_ref, a_ref, b_ref, dist_ref, rinv_ref, minc_ref):
    h = jnp.maximum(
        jnp.dot(x_ref[...], wemb_ref[...], preferred_element_type=jnp.float32)
        + bemb_ref[...], 0.0)
    ab = jnp.dot(h, w1ab_ref[...], preferred_element_type=jnp.float32)
    a_ref[0:_N, :] = ab[:, :_H] + b1_ref[...]
    b_ref[...] = ab[:, _H:]
    dist_ref[...] = jnp.sqrt(sq_ref[...] + 1e-12)
    cnt = cntp_ref[0, :_N, 0:1] + cntp_ref[1, :_N, 0:1]
    rinv_ref[...] = 1.0 / jnp.maximum(cnt, 1.0)
    minc_ref[...] = jnp.minimum(cnt, 1.0)


def _tc_layer_body(p_ref, rinv_ref, minc_ref, w2_ref, b2_ref, w1ab_ref, b1_ref,
                   a_ref, b_ref):
    u = p_ref[0, :_N, :] + p_ref[1, :_N, :]
    h = jnp.maximum(
        jnp.dot(u, w2_ref[...], preferred_element_type=jnp.float32)
        * rinv_ref[...] + b2_ref[...] * minc_ref[...], 0.0)
    ab = jnp.dot(h, w1ab_ref[...], preferred_element_type=jnp.float32)
    a_ref[0:_N, :] = ab[:, :_H] + b1_ref[...]
    b_ref[...] = ab[:, _H:]


def _tc_final_body(p_ref, rinv_ref, minc_ref, w2_ref, b2_ref, wout_ref,
                   bout_ref, out_ref):
    u = p_ref[0, :_N, :] + p_ref[1, :_N, :]
    h = jnp.maximum(
        jnp.dot(u, w2_ref[...], preferred_element_type=jnp.float32)
        * rinv_ref[...] + b2_ref[...] * minc_ref[...], 0.0)
    out_ref[...] = (jnp.dot(h, wout_ref[...], preferred_element_type=jnp.float32)
                    + bout_ref[...])


_f32 = jnp.float32

_tc_embed = pl.pallas_call(
    _tc_embed_body,
    out_shape=(
        jax.ShapeDtypeStruct((_NP, _H), _f32),
        jax.ShapeDtypeStruct((_N, _H), _f32),
        jax.ShapeDtypeStruct((_NW, _NCHUNK, _K), _f32),
        jax.ShapeDtypeStruct((_N, 1), _f32),
        jax.ShapeDtypeStruct((_N, 1), _f32),
    ),
)

_tc_layer = pl.pallas_call(
    _tc_layer_body,
    out_shape=(
        jax.ShapeDtypeStruct((_NP, _H), _f32),
        jax.ShapeDtypeStruct((_N, _H), _f32),
    ),
)

_tc_final = pl.pallas_call(
    _tc_final_body,
    out_shape=jax.ShapeDtypeStruct((_N, 2), _f32),
)


def kernel(x, edge_index, pos, W_emb, b_emb, W1_0, b1_0, W2_0, b2_0,
           W1_1, b1_1, W2_1, b2_1, W1_2, b1_2, W2_2, b2_2, W_out, b_out):
    pad = ((0, 0), (0, _EPWP - _EPW))
    src = jnp.pad(edge_index[0].reshape(_NW, _EPW), pad,
                  constant_values=0).reshape(_NW, _NCHUNK, _K)
    dst = jnp.pad(edge_index[1].reshape(_NW, _EPW), pad,
                  constant_values=_N).reshape(_NW, _NCHUNK, _K)
    posp = jnp.pad(pos, ((0, _NP - _N), (0, 0)))
    posx = jnp.asarray(posp[:, 0])
    posy = jnp.asarray(posp[:, 1])
    posz = jnp.asarray(posp[:, 2])
    zeros16 = jnp.zeros((_NP, 16), _f32)
    zeros64 = jnp.zeros((_NP, _H), _f32)
    ones16 = jnp.ones((_K, 16), _f32)

    sq, cntp = _sc_prep(posx, posy, posz, dst, src, zeros16, ones16)

    w1 = [(W1_0, b1_0, W2_0, b2_0), (W1_1, b1_1, W2_1, b2_1),
          (W1_2, b1_2, W2_2, b2_2)]
    w1ab = [jnp.concatenate([W1[:_H], W1[_H:2 * _H]], axis=1)
            for (W1, _, _, _) in w1]
    wc = [W1[2 * _H] for (W1, _, _, _) in w1]
    b1r = [b1.reshape(1, _H) for (_, b1, _, _) in w1]

    a0, b0, distr, rinv, minc = _tc_embed(
        x, W_emb, b_emb.reshape(1, _H), w1ab[0], b1r[0], sq, cntp)

    a, b = a0, b0
    for l in range(3):
        p = _sc_layer(a, b, dst, src, distr, wc[l], zeros64)
        w2, bb2 = w1[l][2], w1[l][3].reshape(1, _H)
        if l < 2:
            a, b = _tc_layer(p, rinv, minc, w2, bb2, w1ab[l + 1], b1r[l + 1])
        else:
            out = _tc_final(p, rinv, minc, w2, bb2, W_out,
                            b_out.reshape(1, 2))
    return out


# speedup vs baseline: 1.7065x; 1.7065x over previous
import functools
import jax
import jax.numpy as jnp
from jax import lax
from jax.experimental import pallas as pl
from jax.experimental.pallas import tpu as pltpu
from jax.experimental.pallas import tpu_sc as plsc

_N = 10000
_E = 320000
_DIN = 128
_H = 64
_NC = 2
_NS = 16
_NW = _NC * _NS
_EPW = _E // _NW
_K = 80
_NCHUNK = _EPW // _K
_NP = 10240
_RPT = _NP // _NS

_mesh = plsc.VectorSubcoreMesh(core_axis_name="c", subcore_axis_name="s")
_sc_params = pltpu.CompilerParams(needs_layout_passes=False,
                                  use_tc_tiling_on_sc=False)


@functools.partial(
    pl.kernel,
    mesh=_mesh,
    compiler_params=_sc_params,
    out_type=(
        jax.ShapeDtypeStruct((_NW, _NCHUNK, _K), jnp.float32),
        jax.ShapeDtypeStruct((_NC, _NP, 16), jnp.float32),
    ),
    scratch_types=[
        pltpu.VMEM((_N,), jnp.float32),
        pltpu.VMEM((_N,), jnp.float32),
        pltpu.VMEM((_N,), jnp.float32),
        pltpu.VMEM((_NCHUNK, _K), jnp.int32),
        pltpu.VMEM((_NCHUNK, _K), jnp.int32),
        pltpu.VMEM((_NCHUNK, _K), jnp.float32),
        pltpu.VMEM((_K, 16), jnp.float32),
        pltpu.VMEM_SHARED((_NP, 16), jnp.float32),
    ],
)
def _sc_prep(posx, posy, posz, dste, srce, zeros16, ones16,
             sq_out, cnt_out, px, py, pz, idx_d, idx_s, sqv, onesv, cnt_sh):
    cid = lax.axis_index("c")
    sid = lax.axis_index("s")
    wid = sid * _NC + cid
    pltpu.sync_copy(posx, px)
    pltpu.sync_copy(posy, py)
    pltpu.sync_copy(posz, pz)
    pltpu.sync_copy(dste.at[wid], idx_d)
    pltpu.sync_copy(srce.at[wid], idx_s)
    pltpu.sync_copy(ones16, onesv)
    pltpu.sync_copy(zeros16.at[pl.ds(sid * _RPT, _RPT)],
                    cnt_sh.at[pl.ds(sid * _RPT, _RPT)])
    plsc.subcore_barrier()

    def chunk(t, carry):
        for g in range(_K // 16):
            i_dst = idx_d[t, pl.ds(g * 16, 16)]
            i_src = idx_s[t, pl.ds(g * 16, 16)]
            dx = plsc.load_gather(px, [i_src]) - plsc.load_gather(px, [i_dst])
            dy = plsc.load_gather(py, [i_src]) - plsc.load_gather(py, [i_dst])
            dz = plsc.load_gather(pz, [i_src]) - plsc.load_gather(pz, [i_dst])
            sqv[t, pl.ds(g * 16, 16)] = dx * dx + dy * dy + dz * dz
        pltpu.sync_copy(onesv, cnt_sh.at[idx_d.at[t]], add=True)
        return carry

    lax.fori_loop(0, _NCHUNK, chunk, 0)
    pltpu.sync_copy(sqv, sq_out.at[wid])
    plsc.subcore_barrier()
    pltpu.sync_copy(cnt_sh.at[pl.ds(sid * _RPT, _RPT)],
                    cnt_out.at[cid, pl.ds(sid * _RPT, _RPT)])


@functools.partial(
    pl.kernel,
    mesh=_mesh,
    compiler_params=_sc_params,
    out_type=jax.ShapeDtypeStruct((_NC, _NP, _H), jnp.float32),
    scratch_types=[
        pltpu.VMEM((_NCHUNK, _K), jnp.int32),
        pltpu.VMEM((_NCHUNK, _K), jnp.int32),
        pltpu.VMEM((_NCHUNK, _K), jnp.float32),
        pltpu.VMEM((2, _K, _H), jnp.float32),
        pltpu.VMEM((2, _K, _H), jnp.float32),
        pltpu.VMEM((2, _K, _H), jnp.float32),
        pltpu.VMEM((_H,), jnp.float32),
        pltpu.VMEM_SHARED((_NP, _H), jnp.float32),
    ] + [pltpu.SemaphoreType.DMA] * 6,
)
def _sc_layer(a_hbm, b_hbm, dste, srce, distr, wc_hbm, zeros64,
              p_out, idx_d, idx_s, dl, ai2, bj2, vv2, wcv, u_sh,
              sem_a0, sem_a1,
              sem_b0, sem_b1,
              sem_s0, sem_s1):
    cid = lax.axis_index("c")
    sid = lax.axis_index("s")
    wid = sid * _NC + cid
    pltpu.sync_copy(dste.at[wid], idx_d)
    pltpu.sync_copy(srce.at[wid], idx_s)
    pltpu.sync_copy(distr.at[wid], dl)
    pltpu.sync_copy(wc_hbm, wcv)
    pltpu.sync_copy(zeros64.at[pl.ds(sid * _RPT, _RPT)],
                    u_sh.at[pl.ds(sid * _RPT, _RPT)])
    plsc.subcore_barrier()
    wc = [wcv[pl.ds(c * 16, 16)] for c in range(_H // 16)]
    sem_a = (sem_a0, sem_a1)
    sem_b = (sem_b0, sem_b1)
    sem_s = (sem_s0, sem_s1)

    def issue_gathers(t, h):
        pltpu.async_copy(a_hbm.at[idx_d.at[t]], ai2.at[h], sem_a[h])
        pltpu.async_copy(b_hbm.at[idx_s.at[t]], bj2.at[h], sem_b[h])

    def wait_gathers(t, h):
        pltpu.make_async_copy(a_hbm.at[idx_d.at[t]], ai2.at[h], sem_a[h]).wait()
        pltpu.make_async_copy(b_hbm.at[idx_s.at[t]], bj2.at[h], sem_b[h]).wait()

    def compute(t, h):
        ai = ai2.at[h]
        bj = bj2.at[h]
        vv = vv2.at[h]

        def edge16(g, c2):
            dvec = dl[t, pl.ds(g * 16, 16)]
            for el in range(16):
                e = g * 16 + el
                for c in range(_H // 16):
                    s = pl.ds(c * 16, 16)
                    vv[e, s] = jnp.maximum(
                        ai[e, s] + bj[e, s] + dvec[el] * wc[c], 0.0)
            return c2

        lax.fori_loop(0, _K // 16, edge16, 0, unroll=True)

    def step(t, h):
        @pl.when(t + 1 < _NCHUNK)
        def _():
            issue_gathers(t + 1, 1 - h)
        wait_gathers(t, h)
        @pl.when(t >= 2)
        def _():
            pltpu.make_async_copy(vv2.at[h], u_sh.at[idx_d.at[t]],
                                  sem_s[h]).wait()
        compute(t, h)
        pltpu.async_copy(vv2.at[h], u_sh.at[idx_d.at[t]], sem_s[h], add=True)

    issue_gathers(0, 0)
    step(0, 0)

    def pair(tt, carry):
        step(2 * tt + 1, 1)
        step(2 * tt + 2, 0)
        return carry

    lax.fori_loop(0, (_NCHUNK - 1) // 2, pair, 0)
    pltpu.make_async_copy(vv2.at[0], u_sh.at[idx_d.at[0]], sem_s0).wait()
    pltpu.make_async_copy(vv2.at[1], u_sh.at[idx_d.at[0]], sem_s1).wait()
    plsc.subcore_barrier()
    pltpu.sync_copy(u_sh.at[pl.ds(sid * _RPT, _RPT)],
                    p_out.at[cid, pl.ds(sid * _RPT, _RPT)])


def _tc_embed_body(x_ref, wemb_ref, bemb_ref, w1ab_ref, b1_ref, sq_ref,
                   cntp_ref, a_ref, b_ref, dist_ref, rinv_ref, minc_ref):
    h = jnp.maximum(
        jnp.dot(x_ref[...], wemb_ref[...], preferred_element_type=jnp.float32)
        + bemb_ref[...], 0.0)
    ab = jnp.dot(h, w1ab_ref[...], preferred_element_type=jnp.float32)
    a_ref[...] = ab[:, :_H] + b1_ref[...]
    b_ref[...] = ab[:, _H:]
    dist_ref[...] = jnp.sqrt(sq_ref[...] + 1e-12)
    cnt = cntp_ref[0, :_N, 0:1] + cntp_ref[1, :_N, 0:1]
    rinv_ref[...] = 1.0 / jnp.maximum(cnt, 1.0)
    minc_ref[...] = jnp.minimum(cnt, 1.0)


def _tc_layer_body(p_ref, rinv_ref, minc_ref, w2_ref, b2_ref, w1ab_ref, b1_ref,
                   a_ref, b_ref):
    u = p_ref[0, :_N, :] + p_ref[1, :_N, :]
    h = jnp.maximum(
        jnp.dot(u, w2_ref[...], preferred_element_type=jnp.float32)
        * rinv_ref[...] + b2_ref[...] * minc_ref[...], 0.0)
    ab = jnp.dot(h, w1ab_ref[...], preferred_element_type=jnp.float32)
    a_ref[...] = ab[:, :_H] + b1_ref[...]
    b_ref[...] = ab[:, _H:]


def _tc_final_body(p_ref, rinv_ref, minc_ref, w2_ref, b2_ref, wout_ref,
                   bout_ref, out_ref):
    u = p_ref[0, :_N, :] + p_ref[1, :_N, :]
    h = jnp.maximum(
        jnp.dot(u, w2_ref[...], preferred_element_type=jnp.float32)
        * rinv_ref[...] + b2_ref[...] * minc_ref[...], 0.0)
    out_ref[...] = (jnp.dot(h, wout_ref[...], preferred_element_type=jnp.float32)
                    + bout_ref[...])


_f32 = jnp.float32

_tc_embed = pl.pallas_call(
    _tc_embed_body,
    out_shape=(
        jax.ShapeDtypeStruct((_N, _H), _f32),
        jax.ShapeDtypeStruct((_N, _H), _f32),
        jax.ShapeDtypeStruct((_NW, _NCHUNK, _K), _f32),
        jax.ShapeDtypeStruct((_N, 1), _f32),
        jax.ShapeDtypeStruct((_N, 1), _f32),
    ),
)

_tc_layer = pl.pallas_call(
    _tc_layer_body,
    out_shape=(
        jax.ShapeDtypeStruct((_N, _H), _f32),
        jax.ShapeDtypeStruct((_N, _H), _f32),
    ),
)

_tc_final = pl.pallas_call(
    _tc_final_body,
    out_shape=jax.ShapeDtypeStruct((_N, 2), _f32),
)


def kernel(x, edge_index, pos, W_emb, b_emb, W1_0, b1_0, W2_0, b2_0,
           W1_1, b1_1, W2_1, b2_1, W1_2, b1_2, W2_2, b2_2, W_out, b_out):
    src = edge_index[0].reshape(_NW, _NCHUNK, _K)
    dst = edge_index[1].reshape(_NW, _NCHUNK, _K)
    posx = jnp.asarray(pos[:, 0])
    posy = jnp.asarray(pos[:, 1])
    posz = jnp.asarray(pos[:, 2])
    zeros16 = jnp.zeros((_NP, 16), _f32)
    zeros64 = jnp.zeros((_NP, _H), _f32)
    ones16 = jnp.ones((_K, 16), _f32)

    sq, cntp = _sc_prep(posx, posy, posz, dst, src, zeros16, ones16)

    w1 = [(W1_0, b1_0, W2_0, b2_0), (W1_1, b1_1, W2_1, b2_1),
          (W1_2, b1_2, W2_2, b2_2)]
    w1ab = [jnp.concatenate([W1[:_H], W1[_H:2 * _H]], axis=1)
            for (W1, _, _, _) in w1]
    wc = [W1[2 * _H] for (W1, _, _, _) in w1]
    b1r = [b1.reshape(1, _H) for (_, b1, _, _) in w1]

    a0, b0, distr, rinv, minc = _tc_embed(
        x, W_emb, b_emb.reshape(1, _H), w1ab[0], b1r[0], sq, cntp)

    a, b = a0, b0
    for l in range(3):
        p = _sc_layer(a, b, dst, src, distr, wc[l], zeros64)
        w2, bb2 = w1[l][2], w1[l][3].reshape(1, _H)
        if l < 2:
            a, b = _tc_layer(p, rinv, minc, w2, bb2, w1ab[l + 1], b1r[l + 1])
        else:
            out = _tc_final(p, rinv, minc, w2, bb2, W_out,
                            b_out.reshape(1, 2))
    return out
